# baseline (speedup 1.0000x reference)
import functools

import numpy as np
import jax
import jax.numpy as jnp
from jax import lax
from jax.experimental import pallas as pl
from jax.experimental.pallas import tpu as pltpu
from jax.experimental.pallas import tpu_sc as plsc

N, D = 10000, 128
NP = 10240
E = 320000
NC, NS = 2, 16
NW = NC * NS
CHUNK = 128
NCHUNK = 80
EPW = CHUNK * NCHUNK
EP = EPW * NW
RPT = NP // NS
LAST = N - (NS - 1) * RPT
BN_EPS = 1e-5

_PAD_SRC = np.arange(EP - E, dtype=np.int32) % 9984
_PAD_DST = N + np.arange(EP - E, dtype=np.int32) % (NP - N)


def _sc_aggregate(x, src_r, dst_r):
  mesh = plsc.VectorSubcoreMesh(
      core_axis_name="c", subcore_axis_name="s",
      num_cores=NC, num_subcores=NS)

  @functools.partial(
      pl.kernel,
      out_type=jax.ShapeDtypeStruct((NC, N, D), jnp.float32),
      mesh=mesh,
      scratch_types=[
          pltpu.VMEM((EPW,), jnp.int32),
          pltpu.VMEM((CHUNK,), jnp.int32),
          pltpu.VMEM((CHUNK,), jnp.int32),
          pltpu.VMEM((CHUNK, D), jnp.float32),
          pltpu.VMEM((CHUNK, D), jnp.float32),
          pltpu.VMEM_SHARED((NP, D), jnp.float32),
          pltpu.SemaphoreType.DMA,
          pltpu.SemaphoreType.DMA,
          pltpu.SemaphoreType.DMA,
          pltpu.SemaphoreType.DMA,
          pltpu.SemaphoreType.DMA,
          pltpu.SemaphoreType.DMA,
          pltpu.SemaphoreType.DMA,
      ],
  )
  def agg_kernel(x_hbm, src_hbm, dst_hbm, out_hbm, src_v,
                 idst_a, idst_b, rows_a, rows_b, acc,
                 gsem_a, gsem_b, dsem_a, dsem_b,
                 scsem_a, scsem_b, ssem):
    c = lax.axis_index("c")
    s = lax.axis_index("s")
    w = c * NS + s
    pltpu.async_copy(src_hbm.at[pl.ds(w * EPW, EPW)], src_v, ssem).wait()

    def gather_start(j, buf, sem):
      pltpu.async_copy(x_hbm.at[src_v.at[pl.ds(j * CHUNK, CHUNK)]], buf, sem)

    def gather_wait(j, buf, sem):
      pltpu.make_async_copy(
          x_hbm.at[src_v.at[pl.ds(j * CHUNK, CHUNK)]], buf, sem).wait()

    def idx_start(j, buf, sem):
      pltpu.async_copy(dst_hbm.at[pl.ds(w * EPW + j * CHUNK, CHUNK)], buf, sem)

    def idx_wait(j, buf, sem):
      pltpu.make_async_copy(
          dst_hbm.at[pl.ds(w * EPW + j * CHUNK, CHUNK)], buf, sem).wait()

    idx_start(0, idst_a, dsem_a)
    idx_start(1, idst_b, dsem_b)
    gather_start(0, rows_a, gsem_a)
    gather_start(1, rows_b, gsem_b)

    @pl.when(s < NS - 1)
    def _():
      pltpu.sync_copy(x_hbm.at[pl.ds(s * RPT, RPT)],
                      acc.at[pl.ds(s * RPT, RPT)])
    @pl.when(s == NS - 1)
    def _():
      pltpu.sync_copy(x_hbm.at[pl.ds((NS - 1) * RPT, LAST)],
                      acc.at[pl.ds((NS - 1) * RPT, LAST)])
    plsc.subcore_barrier()

    def body(i, carry):
      b = 2 * i
      gather_wait(b, rows_a, gsem_a)
      idx_wait(b, idst_a, dsem_a)
      pltpu.async_copy(rows_a, acc.at[idst_a], scsem_a, add=True)
      gather_wait(b + 1, rows_b, gsem_b)
      idx_wait(b + 1, idst_b, dsem_b)
      pltpu.async_copy(rows_b, acc.at[idst_b], scsem_b, add=True)
      pltpu.make_async_copy(rows_a, acc.at[idst_a], scsem_a).wait()
      idx_start(b + 2, idst_a, dsem_a)
      gather_start(b + 2, rows_a, gsem_a)
      pltpu.make_async_copy(rows_b, acc.at[idst_b], scsem_b).wait()
      idx_start(b + 3, idst_b, dsem_b)
      gather_start(b + 3, rows_b, gsem_b)
      return carry

    lax.fori_loop(0, NCHUNK // 2 - 1, body, 0)
    gather_wait(NCHUNK - 2, rows_a, gsem_a)
    idx_wait(NCHUNK - 2, idst_a, dsem_a)
    pltpu.sync_copy(rows_a, acc.at[idst_a], add=True)
    gather_wait(NCHUNK - 1, rows_b, gsem_b)
    idx_wait(NCHUNK - 1, idst_b, dsem_b)
    pltpu.sync_copy(rows_b, acc.at[idst_b], add=True)

    plsc.subcore_barrier()
    @pl.when(s < NS - 1)
    def _():
      pltpu.sync_copy(acc.at[pl.ds(s * RPT, RPT)],
                      out_hbm.at[c, pl.ds(s * RPT, RPT)])
    @pl.when(s == NS - 1)
    def _():
      pltpu.sync_copy(acc.at[pl.ds((NS - 1) * RPT, LAST)],
                      out_hbm.at[c, pl.ds((NS - 1) * RPT, LAST)])

  return agg_kernel(x, src_r, dst_r)


def _mlp_body(eps_ref, x_ref, p_ref, w1_ref, b1_ref, g1_ref,
              be1_ref, w2_ref, b2_ref, g2_ref, be2_ref, o_ref):
  eps = eps_ref[0]
  h = p_ref[0] + p_ref[1] + (eps - 1.0) * x_ref[...]
  h = lax.dot_general(h, w1_ref[...], (((1,), (1,)), ((), ())),
                      preferred_element_type=jnp.float32)
  h = h + b1_ref[...]
  mu = jnp.mean(h, axis=0, keepdims=True)
  var = jnp.mean((h - mu) ** 2, axis=0, keepdims=True)
  h = (h - mu) / jnp.sqrt(var + BN_EPS) * g1_ref[...] + be1_ref[...]
  h = jnp.maximum(h, 0.0)
  h = lax.dot_general(h, w2_ref[...], (((1,), (1,)), ((), ())),
                      preferred_element_type=jnp.float32)
  h = h + b2_ref[...]
  mu = jnp.mean(h, axis=0, keepdims=True)
  var = jnp.mean((h - mu) ** 2, axis=0, keepdims=True)
  h = (h - mu) / jnp.sqrt(var + BN_EPS) * g2_ref[...] + be2_ref[...]
  o_ref[...] = jnp.maximum(h, 0.0)


def _mlp(eps, x, parts, W1, b1, g1, be1, W2, b2, g2, be2):
  vspec = pl.BlockSpec(memory_space=pltpu.VMEM)
  return pl.pallas_call(
      _mlp_body,
      out_shape=jax.ShapeDtypeStruct((N, D), jnp.float32),
      in_specs=[pl.BlockSpec(memory_space=pltpu.SMEM)] + [vspec] * 10,
      out_specs=vspec,
  )(eps, x, parts, W1, b1, g1, be1, W2, b2, g2, be2)


def kernel(x, edge_index, eps, W1, b1, g1, be1, W2, b2, g2, be2):
  ei = edge_index.astype(jnp.int32)
  src = jnp.concatenate([ei[0], jnp.asarray(_PAD_SRC)])
  dst = jnp.concatenate([ei[1], jnp.asarray(_PAD_DST)])
  parts = _sc_aggregate(x, src, dst)
  row = lambda v: v.reshape(1, D)
  return _mlp(eps.reshape(1), x, parts,
              W1, row(b1), row(g1), row(be1), W2, row(b2), row(g2), row(be2))


# speedup vs baseline: 1.2421x; 1.2421x over previous
import functools

import numpy as np
import jax
import jax.numpy as jnp
from jax import lax
from jax.experimental import pallas as pl
from jax.experimental.pallas import tpu as pltpu
from jax.experimental.pallas import tpu_sc as plsc

N, D = 10000, 128
NP = 10240
E = 320000
NC, NS = 2, 16
NW = NC * NS
CHUNK = 128
NCHUNK = 80
EPW = CHUNK * NCHUNK
EP = EPW * NW
RPT = NP // NS
LAST = N - (NS - 1) * RPT
BN_EPS = 1e-5

_PAD_SRC = np.arange(EP - E, dtype=np.int32) % 9984
_PAD_DST = N + np.arange(EP - E, dtype=np.int32) % (NP - N)


def _sc_aggregate(x, src_r, dst_r):
  mesh = plsc.VectorSubcoreMesh(
      core_axis_name="c", subcore_axis_name="s",
      num_cores=NC, num_subcores=NS)

  @functools.partial(
      pl.kernel,
      out_type=jax.ShapeDtypeStruct((NC, N, D), jnp.float32),
      mesh=mesh,
      scratch_types=[
          pltpu.VMEM((EPW,), jnp.int32),
          pltpu.VMEM((CHUNK,), jnp.int32),
          pltpu.VMEM((CHUNK,), jnp.int32),
          pltpu.VMEM((CHUNK, D), jnp.float32),
          pltpu.VMEM((CHUNK, D), jnp.float32),
          pltpu.VMEM_SHARED((NP, D), jnp.float32),
          pltpu.SemaphoreType.DMA,
          pltpu.SemaphoreType.DMA,
          pltpu.SemaphoreType.DMA,
          pltpu.SemaphoreType.DMA,
          pltpu.SemaphoreType.DMA,
      ],
  )
  def agg_kernel(x_hbm, src_hbm, dst_hbm, out_hbm, src_v,
                 idst_a, idst_b, rows_a, rows_b, acc,
                 gsem_a, gsem_b, dsem_a, dsem_b, ssem):
    c = lax.axis_index("c")
    s = lax.axis_index("s")
    w = c * NS + s
    stage = pltpu.async_copy(src_hbm.at[pl.ds(w * EPW, EPW)], src_v, ssem)

    def gather_start(j, buf, sem):
      pltpu.async_copy(x_hbm.at[src_v.at[pl.ds(j * CHUNK, CHUNK)]], buf, sem)

    def gather_wait(j, buf, sem):
      pltpu.make_async_copy(
          x_hbm.at[src_v.at[pl.ds(j * CHUNK, CHUNK)]], buf, sem).wait()

    def idx_start(j, buf, sem):
      pltpu.async_copy(dst_hbm.at[pl.ds(w * EPW + j * CHUNK, CHUNK)], buf, sem)

    def idx_wait(j, buf, sem):
      pltpu.make_async_copy(
          dst_hbm.at[pl.ds(w * EPW + j * CHUNK, CHUNK)], buf, sem).wait()

    @pl.when(s < NS - 1)
    def _():
      pltpu.sync_copy(x_hbm.at[pl.ds(s * RPT, RPT)],
                      acc.at[pl.ds(s * RPT, RPT)])
    @pl.when(s == NS - 1)
    def _():
      pltpu.sync_copy(x_hbm.at[pl.ds((NS - 1) * RPT, LAST)],
                      acc.at[pl.ds((NS - 1) * RPT, LAST)])
    stage.wait()
    plsc.subcore_barrier()

    idx_start(0, idst_a, dsem_a)
    idx_start(1, idst_b, dsem_b)
    gather_start(0, rows_a, gsem_a)
    gather_start(1, rows_b, gsem_b)

    def body(i, carry):
      b = 2 * i
      gather_wait(b, rows_a, gsem_a)
      idx_wait(b, idst_a, dsem_a)
      pltpu.sync_copy(rows_a, acc.at[idst_a], add=True)
      idx_start(b + 2, idst_a, dsem_a)
      gather_start(b + 2, rows_a, gsem_a)
      gather_wait(b + 1, rows_b, gsem_b)
      idx_wait(b + 1, idst_b, dsem_b)
      pltpu.sync_copy(rows_b, acc.at[idst_b], add=True)
      idx_start(b + 3, idst_b, dsem_b)
      gather_start(b + 3, rows_b, gsem_b)
      return carry

    lax.fori_loop(0, NCHUNK // 2 - 1, body, 0)
    gather_wait(NCHUNK - 2, rows_a, gsem_a)
    idx_wait(NCHUNK - 2, idst_a, dsem_a)
    pltpu.sync_copy(rows_a, acc.at[idst_a], add=True)
    gather_wait(NCHUNK - 1, rows_b, gsem_b)
    idx_wait(NCHUNK - 1, idst_b, dsem_b)
    pltpu.sync_copy(rows_b, acc.at[idst_b], add=True)

    plsc.subcore_barrier()
    @pl.when(s < NS - 1)
    def _():
      pltpu.sync_copy(acc.at[pl.ds(s * RPT, RPT)],
                      out_hbm.at[c, pl.ds(s * RPT, RPT)])
    @pl.when(s == NS - 1)
    def _():
      pltpu.sync_copy(acc.at[pl.ds((NS - 1) * RPT, LAST)],
                      out_hbm.at[c, pl.ds((NS - 1) * RPT, LAST)])

  return agg_kernel(x, src_r, dst_r)


def _mlp_body(eps_ref, x_ref, p_ref, w1_ref, b1_ref, g1_ref,
              be1_ref, w2_ref, b2_ref, g2_ref, be2_ref, o_ref):
  eps = eps_ref[0]
  h = p_ref[0] + p_ref[1] + (eps - 1.0) * x_ref[...]
  h = lax.dot_general(h, w1_ref[...], (((1,), (1,)), ((), ())),
                      preferred_element_type=jnp.float32)
  h = h + b1_ref[...]
  mu = jnp.mean(h, axis=0, keepdims=True)
  var = jnp.mean((h - mu) ** 2, axis=0, keepdims=True)
  h = (h - mu) / jnp.sqrt(var + BN_EPS) * g1_ref[...] + be1_ref[...]
  h = jnp.maximum(h, 0.0)
  h = lax.dot_general(h, w2_ref[...], (((1,), (1,)), ((), ())),
                      preferred_element_type=jnp.float32)
  h = h + b2_ref[...]
  mu = jnp.mean(h, axis=0, keepdims=True)
  var = jnp.mean((h - mu) ** 2, axis=0, keepdims=True)
  h = (h - mu) / jnp.sqrt(var + BN_EPS) * g2_ref[...] + be2_ref[...]
  o_ref[...] = jnp.maximum(h, 0.0)


def _mlp(eps, x, parts, W1, b1, g1, be1, W2, b2, g2, be2):
  vspec = pl.BlockSpec(memory_space=pltpu.VMEM)
  return pl.pallas_call(
      _mlp_body,
      out_shape=jax.ShapeDtypeStruct((N, D), jnp.float32),
      in_specs=[pl.BlockSpec(memory_space=pltpu.SMEM)] + [vspec] * 10,
      out_specs=vspec,
  )(eps, x, parts, W1, b1, g1, be1, W2, b2, g2, be2)


def kernel(x, edge_index, eps, W1, b1, g1, be1, W2, b2, g2, be2):
  ei = edge_index.astype(jnp.int32)
  src = jnp.concatenate([ei[0], jnp.asarray(_PAD_SRC)])
  dst = jnp.concatenate([ei[1], jnp.asarray(_PAD_DST)])
  parts = _sc_aggregate(x, src, dst)
  row = lambda v: v.reshape(1, D)
  return _mlp(eps.reshape(1), x, parts,
              W1, row(b1), row(g1), row(be1), W2, row(b2), row(g2), row(be2))
